# Initial kernel scaffold; baseline (speedup 1.0000x reference)
import jax
import jax.numpy as jnp
from jax.experimental import pallas as pl


def kernel(x, edge_index, conv_W, conv_b, lin1_W, lin1_b, lin2_W, lin2_b, lin3_W, lin3_b):
    raise NotImplementedError("write your pallas kernel here")


# speedup vs baseline: 13.4640x; 13.4640x over previous
import functools

import jax
import jax.numpy as jnp
from jax import lax
from jax.experimental import pallas as pl
from jax.experimental.pallas import tpu as pltpu
from jax.experimental.pallas import tpu_sc as plsc

N = 10000
E = 320000
D = 128
M = 256

NC = 2
NS = 16
NW = NC * NS
C = 128
CH = 80
EPAD = NW * CH * C
NP = 10240
DUMP = NP - 8
RPT = NP // NS

_MESH = plsc.VectorSubcoreMesh(
    core_axis_name="c", subcore_axis_name="s", num_cores=NC, num_subcores=NS)


@functools.partial(
    pl.kernel,
    out_type=jax.ShapeDtypeStruct((NC, NP, 16), jnp.float32),
    mesh=_MESH,
    scratch_types=[
        pltpu.VMEM_SHARED((NP, 16), jnp.float32),
        pltpu.VMEM((8, C), jnp.int32),
        pltpu.VMEM((C, 16), jnp.float32),
    ],
)
def _sc_deg(edges, out, deg_tbl, dstb, ones):
    c = lax.axis_index("c")
    s = lax.axis_index("s")
    w = c * NS + s

    def fill(val):
        def body(i, _):
            ones[i, :] = jnp.full((16,), val, jnp.float32)
            return 0
        lax.fori_loop(0, C, body, 0)

    fill(0.0)
    for k in range(RPT // C):
        pltpu.sync_copy(ones, deg_tbl.at[pl.ds(s * RPT + k * C, C)])
    plsc.subcore_barrier()
    fill(1.0)

    def group(t, _):
        pltpu.sync_copy(edges.at[1, pl.ds(w * CH + 8 * t, 8)], dstb)
        for u in range(8):
            pltpu.sync_copy(ones, deg_tbl.at[dstb.at[u]], add=True)
        return 0
    lax.fori_loop(0, CH // 8, group, 0)

    plsc.subcore_barrier()
    pltpu.sync_copy(deg_tbl.at[pl.ds(s * RPT, RPT)],
                    out.at[c, pl.ds(s * RPT, RPT)])


@functools.partial(
    pl.kernel,
    out_type=jax.ShapeDtypeStruct((NC, NP, D), jnp.float32),
    mesh=_MESH,
    scratch_types=[
        pltpu.VMEM_SHARED((NP, D), jnp.float32),
        pltpu.VMEM((8, C), jnp.int32),
        pltpu.VMEM((8, C), jnp.int32),
        pltpu.VMEM((2, C, D), jnp.float32),
        pltpu.SemaphoreType.DMA,
        pltpu.SemaphoreType.DMA,
    ],
)
def _sc_scatter(edges, y, out, acc, srcb, dstb, rows, sem0, sem1):
    c = lax.axis_index("c")
    s = lax.axis_index("s")
    w = c * NS + s
    sems = (sem0, sem1)

    def fill_zeros(i, _):
        for k in range(D // 16):
            rows[0, i, pl.ds(k * 16, 16)] = jnp.zeros((16,), jnp.float32)
        return 0
    lax.fori_loop(0, C, fill_zeros, 0)
    for k in range(RPT // C):
        pltpu.sync_copy(rows.at[0], acc.at[pl.ds(s * RPT + k * C, C)])
    plsc.subcore_barrier()

    def gather_start(u, b):
        pltpu.async_copy(y.at[srcb.at[u]], rows.at[b], sems[b])

    def gather_wait(u, b):
        pltpu.make_async_copy(y.at[srcb.at[u]], rows.at[b], sems[b]).wait()

    def scatter(u, b):
        pltpu.sync_copy(rows.at[b], acc.at[dstb.at[u]], add=True)

    def group(t, _):
        pltpu.sync_copy(edges.at[0, pl.ds(w * CH + 8 * t, 8)], srcb)
        pltpu.sync_copy(edges.at[1, pl.ds(w * CH + 8 * t, 8)], dstb)
        gather_start(0, 0)
        for u in range(8):
            if u + 1 < 8:
                gather_start(u + 1, (u + 1) % 2)
            gather_wait(u, u % 2)
            scatter(u, u % 2)
        return 0
    lax.fori_loop(0, CH // 8, group, 0)

    plsc.subcore_barrier()
    pltpu.sync_copy(acc.at[pl.ds(s * RPT, RPT)],
                    out.at[c, pl.ds(s * RPT, RPT)])


def _tc_prep_body(x_ref, w_ref, degp_ref, y_ref, dinv_ref):
    deg = degp_ref[0, :, 0:1] + degp_ref[1, :, 0:1] + 1.0
    dinv = lax.rsqrt(deg)
    xw = jnp.dot(x_ref[...], w_ref[...], preferred_element_type=jnp.float32)
    y_ref[...] = dinv * xw
    dinv_ref[...] = dinv


def _tc_prep(x, conv_W, deg_parts):
    BN = 2000
    grid = (N // BN,)
    return pl.pallas_call(
        _tc_prep_body,
        grid=grid,
        in_specs=[
            pl.BlockSpec((BN, D), lambda i: (i, 0)),
            pl.BlockSpec((D, D), lambda i: (0, 0)),
            pl.BlockSpec((NC, BN, 16), lambda i: (0, i, 0)),
        ],
        out_specs=[
            pl.BlockSpec((BN, D), lambda i: (i, 0)),
            pl.BlockSpec((BN, 1), lambda i: (i, 0)),
        ],
        out_shape=[
            jax.ShapeDtypeStruct((N, D), jnp.float32),
            jax.ShapeDtypeStruct((N, 1), jnp.float32),
        ],
    )(x, conv_W, deg_parts)


def _tc_head_body(x_ref, y_ref, dinv_ref, acc_ref, cb_ref,
                  w1_ref, b1_ref, w2_ref, b2_ref, w3_ref, b3_ref, out_ref):
    g = dinv_ref[...] * (acc_ref[0] + acc_ref[1] + y_ref[...]) + cb_ref[...]
    h = jnp.maximum(g, 0.0) + x_ref[...]
    h1 = jnp.maximum(
        jnp.dot(h, w1_ref[...], preferred_element_type=jnp.float32)
        + b1_ref[...], 0.0)
    h2 = jnp.maximum(
        jnp.dot(h1, w2_ref[...], preferred_element_type=jnp.float32)
        + b2_ref[...], 0.0)
    out_ref[...] = (
        jnp.dot(h2, w3_ref[...], preferred_element_type=jnp.float32)
        + b3_ref[...])


def _tc_head(x, y, dinv, acc, conv_b, lin1_W, lin1_b, lin2_W, lin2_b,
             lin3_W, lin3_b):
    BN = 2000
    grid = (N // BN,)
    full = lambda shape: pl.BlockSpec(shape, lambda i: tuple(0 for _ in shape))
    return pl.pallas_call(
        _tc_head_body,
        grid=grid,
        in_specs=[
            pl.BlockSpec((BN, D), lambda i: (i, 0)),
            pl.BlockSpec((BN, D), lambda i: (i, 0)),
            pl.BlockSpec((BN, 1), lambda i: (i, 0)),
            pl.BlockSpec((NC, BN, D), lambda i: (0, i, 0)),
            full((1, D)),
            full((D, M)),
            full((1, M)),
            full((M, M)),
            full((1, M)),
            full((M, 1)),
            full((1, 1)),
        ],
        out_specs=pl.BlockSpec((BN, 1), lambda i: (i, 0)),
        out_shape=jax.ShapeDtypeStruct((N, 1), jnp.float32),
    )(x, y, dinv, acc, conv_b.reshape(1, D), lin1_W, lin1_b.reshape(1, M),
      lin2_W, lin2_b.reshape(1, M), lin3_W, lin3_b.reshape(1, 1))


def kernel(x, edge_index, conv_W, conv_b, lin1_W, lin1_b, lin2_W, lin2_b,
           lin3_W, lin3_b):
    pad_src = jnp.zeros((EPAD - E,), jnp.int32)
    pad_dst = jnp.full((EPAD - E,), DUMP, jnp.int32)
    edges = jnp.concatenate(
        [edge_index, jnp.stack([pad_src, pad_dst])], axis=1
    ).reshape(2, NW * CH, C)
    deg_parts = _sc_deg(edges)
    y, dinv = _tc_prep(x, conv_W, deg_parts)
    acc = _sc_scatter(edges, y)
    return _tc_head(x, y, dinv, acc, conv_b, lin1_W, lin1_b,
                    lin2_W, lin2_b, lin3_W, lin3_b)


# speedup vs baseline: 33.0762x; 2.4566x over previous
import functools

import jax
import jax.numpy as jnp
from jax import lax
from jax.experimental import pallas as pl
from jax.experimental.pallas import tpu as pltpu
from jax.experimental.pallas import tpu_sc as plsc

N = 10000
E = 320000
D = 128
M = 256

NC = 2
NS = 16
NW = NC * NS
C = 128
CH = 80
EPAD = NW * CH * C
NP = 10240
RPT = NP // NS

_MESH = plsc.VectorSubcoreMesh(
    core_axis_name="c", subcore_axis_name="s", num_cores=NC, num_subcores=NS)


@functools.partial(
    pl.kernel,
    out_type=jax.ShapeDtypeStruct((NC, NP, 16), jnp.float32),
    mesh=_MESH,
    scratch_types=[
        pltpu.VMEM_SHARED((NP, 16), jnp.float32),
        pltpu.VMEM((8, C), jnp.int32),
        pltpu.VMEM((C, 16), jnp.float32),
    ],
)
def _sc_deg(edges, out, deg_tbl, dstb, ones):
    c = lax.axis_index("c")
    s = lax.axis_index("s")
    w = c * NS + s

    def fill(val):
        def body(i, _):
            ones[i, :] = jnp.full((16,), val, jnp.float32)
            return 0
        lax.fori_loop(0, C, body, 0)

    fill(0.0)
    for k in range(RPT // C):
        pltpu.sync_copy(ones, deg_tbl.at[pl.ds(s * RPT + k * C, C)])
    plsc.subcore_barrier()
    fill(1.0)

    def group(t, _):
        pltpu.sync_copy(edges.at[1, pl.ds(w * CH + 8 * t, 8)], dstb)
        for u in range(8):
            pltpu.sync_copy(ones, deg_tbl.at[dstb.at[u]], add=True)
        return 0
    lax.fori_loop(0, CH // 8, group, 0)

    plsc.subcore_barrier()
    pltpu.sync_copy(deg_tbl.at[pl.ds(s * RPT, RPT)],
                    out.at[c, pl.ds(s * RPT, RPT)])


@functools.partial(
    pl.kernel,
    out_type=jax.ShapeDtypeStruct((NC, NP, D), jnp.float32),
    mesh=_MESH,
    scratch_types=[
        pltpu.VMEM_SHARED((NP, D), jnp.float32),
        pltpu.VMEM((8, C), jnp.int32),
        pltpu.VMEM((8, C), jnp.int32),
        pltpu.VMEM((2, C, D), jnp.float32),
        pltpu.SemaphoreType.DMA,
        pltpu.SemaphoreType.DMA,
    ],
)
def _sc_scatter(edges, y, out, acc, srcb, dstb, rows, sem0, sem1):
    c = lax.axis_index("c")
    s = lax.axis_index("s")
    w = c * NS + s
    sems = (sem0, sem1)

    def fill_zeros(i, _):
        for k in range(D // 16):
            rows[0, i, pl.ds(k * 16, 16)] = jnp.zeros((16,), jnp.float32)
        return 0
    lax.fori_loop(0, C, fill_zeros, 0)
    for k in range(RPT // C):
        pltpu.sync_copy(rows.at[0], acc.at[pl.ds(s * RPT + k * C, C)])
    plsc.subcore_barrier()

    def gather_start(u, b):
        pltpu.async_copy(y.at[srcb.at[u]], rows.at[b], sems[b])

    def gather_wait(u, b):
        pltpu.make_async_copy(y.at[srcb.at[u]], rows.at[b], sems[b]).wait()

    def scatter(u, b):
        pltpu.sync_copy(rows.at[b], acc.at[dstb.at[u]], add=True)

    def group(t, _):
        pltpu.sync_copy(edges.at[0, pl.ds(w * CH + 8 * t, 8)], srcb)
        pltpu.sync_copy(edges.at[1, pl.ds(w * CH + 8 * t, 8)], dstb)
        gather_start(0, 0)
        for u in range(8):
            if u + 1 < 8:
                gather_start(u + 1, (u + 1) % 2)
            gather_wait(u, u % 2)
            scatter(u, u % 2)
        return 0
    lax.fori_loop(0, CH // 8, group, 0)

    plsc.subcore_barrier()
    pltpu.sync_copy(acc.at[pl.ds(s * RPT, RPT)],
                    out.at[c, pl.ds(s * RPT, RPT)])


def _tc_prep_body(x_ref, w_ref, degp_ref, y_ref, dinv_ref):
    deg = degp_ref[0, :, 0:1] + degp_ref[1, :, 0:1] + 1.0
    dinv = lax.rsqrt(deg)
    xw = jnp.dot(x_ref[...], w_ref[...], preferred_element_type=jnp.float32)
    y_ref[...] = dinv * xw
    dinv_ref[...] = dinv


def _tc_prep(x, conv_W, deg_parts):
    BN = 2000
    grid = (N // BN,)
    return pl.pallas_call(
        _tc_prep_body,
        grid=grid,
        in_specs=[
            pl.BlockSpec((BN, D), lambda i: (i, 0)),
            pl.BlockSpec((D, D), lambda i: (0, 0)),
            pl.BlockSpec((NC, BN, 16), lambda i: (0, i, 0)),
        ],
        out_specs=[
            pl.BlockSpec((BN, D), lambda i: (i, 0)),
            pl.BlockSpec((BN, 1), lambda i: (i, 0)),
        ],
        out_shape=[
            jax.ShapeDtypeStruct((N, D), jnp.float32),
            jax.ShapeDtypeStruct((N, 1), jnp.float32),
        ],
    )(x, conv_W, deg_parts)


def _tc_head_body(x_ref, y_ref, dinv_ref, acc_ref, cb_ref,
                  w1_ref, b1_ref, w2_ref, b2_ref, w3_ref, b3_ref, out_ref):
    g = dinv_ref[...] * (acc_ref[0] + acc_ref[1] + y_ref[...]) + cb_ref[...]
    h = jnp.maximum(g, 0.0) + x_ref[...]
    h1 = jnp.maximum(
        jnp.dot(h, w1_ref[...], preferred_element_type=jnp.float32)
        + b1_ref[...], 0.0)
    h2 = jnp.maximum(
        jnp.dot(h1, w2_ref[...], preferred_element_type=jnp.float32)
        + b2_ref[...], 0.0)
    out_ref[...] = (
        jnp.dot(h2, w3_ref[...], preferred_element_type=jnp.float32)
        + b3_ref[...])


def _tc_head(x, y, dinv, acc, conv_b, lin1_W, lin1_b, lin2_W, lin2_b,
             lin3_W, lin3_b):
    BN = 2000
    grid = (N // BN,)
    full = lambda shape: pl.BlockSpec(shape, lambda i: tuple(0 for _ in shape))
    return pl.pallas_call(
        _tc_head_body,
        grid=grid,
        in_specs=[
            pl.BlockSpec((BN, D), lambda i: (i, 0)),
            pl.BlockSpec((BN, D), lambda i: (i, 0)),
            pl.BlockSpec((BN, 1), lambda i: (i, 0)),
            pl.BlockSpec((NC, BN, D), lambda i: (0, i, 0)),
            full((1, D)),
            full((D, M)),
            full((1, M)),
            full((M, M)),
            full((1, M)),
            full((M, 1)),
            full((1, 1)),
        ],
        out_specs=pl.BlockSpec((BN, 1), lambda i: (i, 0)),
        out_shape=jax.ShapeDtypeStruct((N, 1), jnp.float32),
    )(x, y, dinv, acc, conv_b.reshape(1, D), lin1_W, lin1_b.reshape(1, M),
      lin2_W, lin2_b.reshape(1, M), lin3_W, lin3_b.reshape(1, 1))


def kernel(x, edge_index, conv_W, conv_b, lin1_W, lin1_b, lin2_W, lin2_b,
           lin3_W, lin3_b):
    ar = jnp.arange(EPAD - E, dtype=jnp.int32)
    pad_src = (ar * 997) % N
    pad_dst = N + (ar % (NP - N))
    edges = jnp.concatenate(
        [edge_index, jnp.stack([pad_src, pad_dst])], axis=1
    ).reshape(2, NW * CH, C)
    deg_parts = _sc_deg(edges)
    y, dinv = _tc_prep(x, conv_W, deg_parts)
    acc = _sc_scatter(edges, y)
    return _tc_head(x, y, dinv, acc, conv_b, lin1_W, lin1_b,
                    lin2_W, lin2_b, lin3_W, lin3_b)


# speedup vs baseline: 39.4943x; 1.1940x over previous
import functools

import jax
import jax.numpy as jnp
from jax import lax
from jax.experimental import pallas as pl
from jax.experimental.pallas import tpu as pltpu
from jax.experimental.pallas import tpu_sc as plsc

N = 10000
E = 320000
D = 128
M = 256

NC = 2
NS = 16
NW = NC * NS
EPAD = 327680
NP = 10240
RPT = NP // NS

_MESH = plsc.VectorSubcoreMesh(
    core_axis_name="c", subcore_axis_name="s", num_cores=NC, num_subcores=NS)

DC = 128
DCH = 80
DNG = DCH // 8
SC_ = 64
SCH = 160
SNG = SCH // 8


@functools.partial(
    pl.kernel,
    out_type=jax.ShapeDtypeStruct((NC, NP, 16), jnp.float32),
    mesh=_MESH,
    scratch_types=[
        pltpu.VMEM_SHARED((NP, 16), jnp.float32),
        pltpu.VMEM((2, 8, DC), jnp.int32),
        pltpu.VMEM((DC, 16), jnp.float32),
        pltpu.SemaphoreType.DMA,
        pltpu.SemaphoreType.DMA,
    ],
)
def _sc_deg(edges, out, deg_tbl, dstb, ones, isem, asem):
    c = lax.axis_index("c")
    s = lax.axis_index("s")
    w = c * NS + s

    def fill(val):
        def body(i, _):
            ones[i, :] = jnp.full((16,), val, jnp.float32)
            return 0
        lax.fori_loop(0, DC, body, 0)

    fill(0.0)
    for k in range(RPT // DC):
        pltpu.sync_copy(ones, deg_tbl.at[pl.ds(s * RPT + k * DC, DC)])
    plsc.subcore_barrier()
    fill(1.0)

    def idx_copy(t, ib):
        return pltpu.make_async_copy(
            edges.at[1, pl.ds(w * DCH + 8 * t, 8)], dstb.at[ib], isem)

    def add_copy(ib, u):
        return pltpu.make_async_copy(
            ones, deg_tbl.at[dstb.at[ib].at[u]], asem)

    idx_copy(0, 0).start()
    idx_copy(0, 0).wait()

    def group(t, ib):
        @pl.when(t >= 1)
        def _():
            for u in range(8):
                add_copy(ib ^ 1, u).wait()

        @pl.when(t + 1 < DNG)
        def _():
            idx_copy(t + 1, ib ^ 1).start()

        @pl.when(t >= 1)
        def _():
            idx_copy(t, ib).wait()
        for u in range(8):
            add_copy(ib, u).start(add=True)

    def pair(i, _):
        group(2 * i, 0)
        group(2 * i + 1, 1)
        return 0
    lax.fori_loop(0, DNG // 2, pair, 0)
    for u in range(8):
        add_copy(1, u).wait()

    plsc.subcore_barrier()
    pltpu.sync_copy(deg_tbl.at[pl.ds(s * RPT, RPT)],
                    out.at[c, pl.ds(s * RPT, RPT)])


@functools.partial(
    pl.kernel,
    out_type=jax.ShapeDtypeStruct((NC, NP, D), jnp.float32),
    mesh=_MESH,
    scratch_types=[
        pltpu.VMEM_SHARED((NP, D), jnp.float32),
        pltpu.VMEM((2, 8, SC_), jnp.int32),
        pltpu.VMEM((2, 8, SC_), jnp.int32),
        pltpu.VMEM((4, SC_, D), jnp.float32),
        [pltpu.SemaphoreType.DMA] * 4,
        [pltpu.SemaphoreType.DMA] * 4,
        pltpu.SemaphoreType.DMA,
    ],
)
def _sc_scatter(edges, y, out, acc, srcb, dstb, rows, gsems, ssems, isem):
    c = lax.axis_index("c")
    s = lax.axis_index("s")
    w = c * NS + s

    def fill_zeros(i, _):
        for k in range(D // 16):
            rows[0, i, pl.ds(k * 16, 16)] = jnp.zeros((16,), jnp.float32)
        return 0
    lax.fori_loop(0, SC_, fill_zeros, 0)
    for k in range(RPT // SC_):
        pltpu.sync_copy(rows.at[0], acc.at[pl.ds(s * RPT + k * SC_, SC_)])
    plsc.subcore_barrier()

    def idx_copies(t, ib):
        return (pltpu.make_async_copy(
                    edges.at[0, pl.ds(w * SCH + 8 * t, 8)], srcb.at[ib], isem),
                pltpu.make_async_copy(
                    edges.at[1, pl.ds(w * SCH + 8 * t, 8)], dstb.at[ib], isem))

    def g_copy(ib, u, b):
        return pltpu.make_async_copy(
            y.at[srcb.at[ib].at[u]], rows.at[b], gsems[b])

    def s_copy(ib, u, b):
        return pltpu.make_async_copy(
            rows.at[b], acc.at[dstb.at[ib].at[u]], ssems[b])

    for cp in idx_copies(0, 0):
        cp.start()
    for cp in idx_copies(0, 0):
        cp.wait()
    g_copy(0, 0, 0).start()
    g_copy(0, 1, 1).start()

    def group(t, ib):
        for u in range(8):
            j = 8 * t + u
            b = u % 4
            b2 = (u + 2) % 4
            pib = ib ^ 1 if u < 2 else ib

            @pl.when(j >= 2)
            def _():
                s_copy(pib, (u + 6) % 8, b2).wait()
            if u == 2:
                @pl.when(t + 1 < SNG)
                def _():
                    for cp in idx_copies(t + 1, ib ^ 1):
                        cp.start()
            nib = ib ^ 1 if u >= 6 else ib

            @pl.when(j + 2 < SCH)
            def _():
                if u == 6:
                    for cp in idx_copies(t + 1, ib ^ 1):
                        cp.wait()
                g_copy(nib, (u + 2) % 8, b2).start()
            g_copy(ib, u, b).wait()
            s_copy(ib, u, b).start(add=True)

    def pair(i, _):
        group(2 * i, 0)
        group(2 * i + 1, 1)
        return 0
    lax.fori_loop(0, SNG // 2, pair, 0)
    s_copy(1, 6, 2).wait()
    s_copy(1, 7, 3).wait()

    plsc.subcore_barrier()
    pltpu.sync_copy(acc.at[pl.ds(s * RPT, RPT)],
                    out.at[c, pl.ds(s * RPT, RPT)])


def _tc_matmul_body(x_ref, w_ref, xw_ref):
    xw_ref[...] = jnp.dot(x_ref[...], w_ref[...],
                          preferred_element_type=jnp.float32)


def _tc_matmul(x, conv_W):
    BN = 2000
    return pl.pallas_call(
        _tc_matmul_body,
        grid=(N // BN,),
        in_specs=[
            pl.BlockSpec((BN, D), lambda i: (i, 0)),
            pl.BlockSpec((D, D), lambda i: (0, 0)),
        ],
        out_specs=pl.BlockSpec((BN, D), lambda i: (i, 0)),
        out_shape=jax.ShapeDtypeStruct((N, D), jnp.float32),
    )(x, conv_W)


def _tc_scale_body(xw_ref, degp_ref, y_ref, dinv_ref):
    deg = degp_ref[0, :, 0:1] + degp_ref[1, :, 0:1] + 1.0
    dinv = lax.rsqrt(deg)
    y_ref[...] = dinv * xw_ref[...]
    dinv_ref[...] = dinv


def _tc_scale(xw, deg_parts):
    BN = 2000
    return pl.pallas_call(
        _tc_scale_body,
        grid=(N // BN,),
        in_specs=[
            pl.BlockSpec((BN, D), lambda i: (i, 0)),
            pl.BlockSpec((NC, BN, 16), lambda i: (0, i, 0)),
        ],
        out_specs=[
            pl.BlockSpec((BN, D), lambda i: (i, 0)),
            pl.BlockSpec((BN, 1), lambda i: (i, 0)),
        ],
        out_shape=[
            jax.ShapeDtypeStruct((N, D), jnp.float32),
            jax.ShapeDtypeStruct((N, 1), jnp.float32),
        ],
    )(xw, deg_parts)


def _tc_head_body(x_ref, y_ref, dinv_ref, acc_ref, cb_ref,
                  w1_ref, b1_ref, w2_ref, b2_ref, w3_ref, b3_ref, out_ref):
    g = dinv_ref[...] * (acc_ref[0] + acc_ref[1] + y_ref[...]) + cb_ref[...]
    h = jnp.maximum(g, 0.0) + x_ref[...]
    h1 = jnp.maximum(
        jnp.dot(h, w1_ref[...], preferred_element_type=jnp.float32)
        + b1_ref[...], 0.0)
    h2 = jnp.maximum(
        jnp.dot(h1, w2_ref[...], preferred_element_type=jnp.float32)
        + b2_ref[...], 0.0)
    out_ref[...] = (
        jnp.dot(h2, w3_ref[...], preferred_element_type=jnp.float32)
        + b3_ref[...])


def _tc_head(x, y, dinv, acc, conv_b, lin1_W, lin1_b, lin2_W, lin2_b,
             lin3_W, lin3_b):
    BN = 2000
    full = lambda shape: pl.BlockSpec(shape, lambda i: tuple(0 for _ in shape))
    return pl.pallas_call(
        _tc_head_body,
        grid=(N // BN,),
        in_specs=[
            pl.BlockSpec((BN, D), lambda i: (i, 0)),
            pl.BlockSpec((BN, D), lambda i: (i, 0)),
            pl.BlockSpec((BN, 1), lambda i: (i, 0)),
            pl.BlockSpec((NC, BN, D), lambda i: (0, i, 0)),
            full((1, D)),
            full((D, M)),
            full((1, M)),
            full((M, M)),
            full((1, M)),
            full((M, 1)),
            full((1, 1)),
        ],
        out_specs=pl.BlockSpec((BN, 1), lambda i: (i, 0)),
        out_shape=jax.ShapeDtypeStruct((N, 1), jnp.float32),
    )(x, y, dinv, acc, conv_b.reshape(1, D), lin1_W, lin1_b.reshape(1, M),
      lin2_W, lin2_b.reshape(1, M), lin3_W, lin3_b.reshape(1, 1))


def kernel(x, edge_index, conv_W, conv_b, lin1_W, lin1_b, lin2_W, lin2_b,
           lin3_W, lin3_b):
    ar = jnp.arange(EPAD - E, dtype=jnp.int32)
    pad_src = (ar * 997) % N
    pad_dst = N + (ar % (NP - N))
    epad = jnp.concatenate([edge_index, jnp.stack([pad_src, pad_dst])], axis=1)
    xw = _tc_matmul(x, conv_W)
    deg_parts = _sc_deg(epad.reshape(2, NW * DCH, DC))
    y, dinv = _tc_scale(xw, deg_parts)
    acc = _sc_scatter(epad.reshape(2, NW * SCH, SC_), y)
    return _tc_head(x, y, dinv, acc, conv_b, lin1_W, lin1_b,
                    lin2_W, lin2_b, lin3_W, lin3_b)


# speedup vs baseline: 39.9987x; 1.0128x over previous
import functools

import jax
import jax.numpy as jnp
from jax import lax
from jax.experimental import pallas as pl
from jax.experimental.pallas import tpu as pltpu
from jax.experimental.pallas import tpu_sc as plsc

N = 10000
E = 320000
D = 128
M = 256

NC = 2
NS = 16
NW = NC * NS
EPAD = 327680
NP = 10240
RPT = NP // NS

_MESH = plsc.VectorSubcoreMesh(
    core_axis_name="c", subcore_axis_name="s", num_cores=NC, num_subcores=NS)

DC = 128
DCH = 80
DNG = DCH // 8
SC_ = 80
SCH = 128
SNG = SCH // 8


@functools.partial(
    pl.kernel,
    out_type=jax.ShapeDtypeStruct((NC, NP, 16), jnp.float32),
    mesh=_MESH,
    scratch_types=[
        pltpu.VMEM_SHARED((NP, 16), jnp.float32),
        pltpu.VMEM((3, 8, DC), jnp.int32),
        pltpu.VMEM((DC, 16), jnp.float32),
        pltpu.SemaphoreType.DMA,
        pltpu.SemaphoreType.DMA,
    ],
)
def _sc_deg(edges, out, deg_tbl, dstb, ones, isem, asem):
    c = lax.axis_index("c")
    s = lax.axis_index("s")
    w = c * NS + s

    def fill(val):
        def body(i, _):
            ones[i, :] = jnp.full((16,), val, jnp.float32)
            return 0
        lax.fori_loop(0, DC, body, 0)

    fill(0.0)
    for k in range(RPT // DC):
        pltpu.sync_copy(ones, deg_tbl.at[pl.ds(s * RPT + k * DC, DC)])
    plsc.subcore_barrier()
    fill(1.0)

    def idx_copy(t, ib):
        return pltpu.make_async_copy(
            edges.at[1, pl.ds(w * DCH + 8 * t, 8)], dstb.at[ib], isem)

    def add_copy(ib, u):
        return pltpu.make_async_copy(
            ones, deg_tbl.at[dstb.at[ib].at[u]], asem)

    idx_copy(0, 0).start()
    idx_copy(0, 0).wait()

    def group(t, ib):
        @pl.when(t >= 2)
        def _():
            for u in range(8):
                add_copy((ib + 1) % 3, u).wait()

        @pl.when(t + 1 < DNG)
        def _():
            idx_copy(t + 1, (ib + 1) % 3).start()

        @pl.when(t >= 1)
        def _():
            idx_copy(t, ib).wait()
        for u in range(8):
            add_copy(ib, u).start(add=True)

    def triple(i, _):
        group(3 * i, 0)
        group(3 * i + 1, 1)
        group(3 * i + 2, 2)
        return 0
    lax.fori_loop(0, DNG // 3, triple, 0)
    group(DNG - 1, (DNG - 1) % 3)
    for u in range(8):
        add_copy((DNG - 2) % 3, u).wait()
        add_copy((DNG - 1) % 3, u).wait()

    plsc.subcore_barrier()
    pltpu.sync_copy(deg_tbl.at[pl.ds(s * RPT, RPT)],
                    out.at[c, pl.ds(s * RPT, RPT)])


@functools.partial(
    pl.kernel,
    out_type=jax.ShapeDtypeStruct((NC, NP, D), jnp.float32),
    mesh=_MESH,
    scratch_types=[
        pltpu.VMEM_SHARED((NP, D), jnp.float32),
        pltpu.VMEM((2, 8, SC_), jnp.int32),
        pltpu.VMEM((2, 8, SC_), jnp.int32),
        pltpu.VMEM((4, SC_, D), jnp.float32),
        [pltpu.SemaphoreType.DMA] * 4,
        [pltpu.SemaphoreType.DMA] * 4,
        pltpu.SemaphoreType.DMA,
    ],
)
def _sc_scatter(edges, y, out, acc, srcb, dstb, rows, gsems, ssems, isem):
    c = lax.axis_index("c")
    s = lax.axis_index("s")
    w = c * NS + s

    def fill_zeros(i, _):
        for k in range(D // 16):
            rows[0, i, pl.ds(k * 16, 16)] = jnp.zeros((16,), jnp.float32)
        return 0
    lax.fori_loop(0, SC_, fill_zeros, 0)
    for k in range(RPT // SC_):
        pltpu.sync_copy(rows.at[0], acc.at[pl.ds(s * RPT + k * SC_, SC_)])
    plsc.subcore_barrier()

    def idx_copies(t, ib):
        return (pltpu.make_async_copy(
                    edges.at[0, pl.ds(w * SCH + 8 * t, 8)], srcb.at[ib], isem),
                pltpu.make_async_copy(
                    edges.at[1, pl.ds(w * SCH + 8 * t, 8)], dstb.at[ib], isem))

    def g_copy(ib, u, b):
        return pltpu.make_async_copy(
            y.at[srcb.at[ib].at[u]], rows.at[b], gsems[b])

    def s_copy(ib, u, b):
        return pltpu.make_async_copy(
            rows.at[b], acc.at[dstb.at[ib].at[u]], ssems[b])

    for cp in idx_copies(0, 0):
        cp.start()
    for cp in idx_copies(0, 0):
        cp.wait()
    g_copy(0, 0, 0).start()
    g_copy(0, 1, 1).start()

    def group(t, ib):
        for u in range(8):
            j = 8 * t + u
            b = u % 4
            b2 = (u + 2) % 4
            pib = ib ^ 1 if u < 2 else ib

            @pl.when(j >= 2)
            def _():
                s_copy(pib, (u + 6) % 8, b2).wait()
            if u == 2:
                @pl.when(t + 1 < SNG)
                def _():
                    for cp in idx_copies(t + 1, ib ^ 1):
                        cp.start()
            nib = ib ^ 1 if u >= 6 else ib

            @pl.when(j + 2 < SCH)
            def _():
                if u == 6:
                    for cp in idx_copies(t + 1, ib ^ 1):
                        cp.wait()
                g_copy(nib, (u + 2) % 8, b2).start()
            g_copy(ib, u, b).wait()
            s_copy(ib, u, b).start(add=True)

    def pair(i, _):
        group(2 * i, 0)
        group(2 * i + 1, 1)
        return 0
    lax.fori_loop(0, SNG // 2, pair, 0)
    s_copy(1, 6, 2).wait()
    s_copy(1, 7, 3).wait()

    plsc.subcore_barrier()
    pltpu.sync_copy(acc.at[pl.ds(s * RPT, RPT)],
                    out.at[c, pl.ds(s * RPT, RPT)])


def _tc_matmul_body(x_ref, w_ref, xw_ref):
    xw_ref[...] = jnp.dot(x_ref[...], w_ref[...],
                          preferred_element_type=jnp.float32)


def _tc_matmul(x, conv_W):
    BN = 2000
    return pl.pallas_call(
        _tc_matmul_body,
        grid=(N // BN,),
        in_specs=[
            pl.BlockSpec((BN, D), lambda i: (i, 0)),
            pl.BlockSpec((D, D), lambda i: (0, 0)),
        ],
        out_specs=pl.BlockSpec((BN, D), lambda i: (i, 0)),
        out_shape=jax.ShapeDtypeStruct((N, D), jnp.float32),
    )(x, conv_W)


def _tc_scale_body(xw_ref, degp_ref, y_ref, dinv_ref):
    deg = degp_ref[0, :, 0:1] + degp_ref[1, :, 0:1] + 1.0
    dinv = lax.rsqrt(deg)
    y_ref[...] = dinv * xw_ref[...]
    dinv_ref[...] = dinv


def _tc_scale(xw, deg_parts):
    BN = 2000
    return pl.pallas_call(
        _tc_scale_body,
        grid=(N // BN,),
        in_specs=[
            pl.BlockSpec((BN, D), lambda i: (i, 0)),
            pl.BlockSpec((NC, BN, 16), lambda i: (0, i, 0)),
        ],
        out_specs=[
            pl.BlockSpec((BN, D), lambda i: (i, 0)),
            pl.BlockSpec((BN, 1), lambda i: (i, 0)),
        ],
        out_shape=[
            jax.ShapeDtypeStruct((N, D), jnp.float32),
            jax.ShapeDtypeStruct((N, 1), jnp.float32),
        ],
    )(xw, deg_parts)


def _tc_head_body(x_ref, y_ref, dinv_ref, acc_ref, cb_ref,
                  w1_ref, b1_ref, w2_ref, b2_ref, w3_ref, b3_ref, out_ref):
    g = dinv_ref[...] * (acc_ref[0] + acc_ref[1] + y_ref[...]) + cb_ref[...]
    h = jnp.maximum(g, 0.0) + x_ref[...]
    h1 = jnp.maximum(
        jnp.dot(h, w1_ref[...], preferred_element_type=jnp.float32)
        + b1_ref[...], 0.0)
    h2 = jnp.maximum(
        jnp.dot(h1, w2_ref[...], preferred_element_type=jnp.float32)
        + b2_ref[...], 0.0)
    out_ref[...] = (
        jnp.dot(h2, w3_ref[...], preferred_element_type=jnp.float32)
        + b3_ref[...])


def _tc_head(x, y, dinv, acc, conv_b, lin1_W, lin1_b, lin2_W, lin2_b,
             lin3_W, lin3_b):
    BN = 2000
    full = lambda shape: pl.BlockSpec(shape, lambda i: tuple(0 for _ in shape))
    return pl.pallas_call(
        _tc_head_body,
        grid=(N // BN,),
        in_specs=[
            pl.BlockSpec((BN, D), lambda i: (i, 0)),
            pl.BlockSpec((BN, D), lambda i: (i, 0)),
            pl.BlockSpec((BN, 1), lambda i: (i, 0)),
            pl.BlockSpec((NC, BN, D), lambda i: (0, i, 0)),
            full((1, D)),
            full((D, M)),
            full((1, M)),
            full((M, M)),
            full((1, M)),
            full((M, 1)),
            full((1, 1)),
        ],
        out_specs=pl.BlockSpec((BN, 1), lambda i: (i, 0)),
        out_shape=jax.ShapeDtypeStruct((N, 1), jnp.float32),
    )(x, y, dinv, acc, conv_b.reshape(1, D), lin1_W, lin1_b.reshape(1, M),
      lin2_W, lin2_b.reshape(1, M), lin3_W, lin3_b.reshape(1, 1))


def kernel(x, edge_index, conv_W, conv_b, lin1_W, lin1_b, lin2_W, lin2_b,
           lin3_W, lin3_b):
    ar = jnp.arange(EPAD - E, dtype=jnp.int32)
    pad_src = (ar * 997) % N
    pad_dst = N + (ar % (NP - N))
    epad = jnp.concatenate([edge_index, jnp.stack([pad_src, pad_dst])], axis=1)
    xw = _tc_matmul(x, conv_W)
    deg_parts = _sc_deg(epad.reshape(2, NW * DCH, DC))
    y, dinv = _tc_scale(xw, deg_parts)
    acc = _sc_scatter(epad.reshape(2, NW * SCH, SC_), y)
    return _tc_head(x, y, dinv, acc, conv_b, lin1_W, lin1_b,
                    lin2_W, lin2_b, lin3_W, lin3_b)


# speedup vs baseline: 40.0625x; 1.0016x over previous
import functools

import jax
import jax.numpy as jnp
from jax import lax
from jax.experimental import pallas as pl
from jax.experimental.pallas import tpu as pltpu
from jax.experimental.pallas import tpu_sc as plsc

N = 10000
E = 320000
D = 128
M = 256

NC = 2
NS = 16
NW = NC * NS
EPAD = 327680
NP = 10240
RPT = NP // NS

_MESH = plsc.VectorSubcoreMesh(
    core_axis_name="c", subcore_axis_name="s", num_cores=NC, num_subcores=NS)

DC = 128
DCH = 80
DNG = DCH // 8
SC_ = 80
SCH = 128
SNG = SCH // 8


@functools.partial(
    pl.kernel,
    out_type=jax.ShapeDtypeStruct((NC, NP, 16), jnp.float32),
    mesh=_MESH,
    scratch_types=[
        pltpu.VMEM_SHARED((NP, 16), jnp.float32),
        pltpu.VMEM((3, 8, DC), jnp.int32),
        pltpu.VMEM((DC, 16), jnp.float32),
        pltpu.SemaphoreType.DMA,
        pltpu.SemaphoreType.DMA,
    ],
)
def _sc_deg(edges, out, deg_tbl, dstb, ones, isem, asem):
    c = lax.axis_index("c")
    s = lax.axis_index("s")
    w = c * NS + s

    def fill(val):
        def body(i, _):
            ones[i, :] = jnp.full((16,), val, jnp.float32)
            return 0
        lax.fori_loop(0, DC, body, 0)

    fill(0.0)
    for k in range(RPT // DC):
        pltpu.make_async_copy(
            ones, deg_tbl.at[pl.ds(s * RPT + k * DC, DC)], asem).start()
    for k in range(RPT // DC):
        pltpu.make_async_copy(
            ones, deg_tbl.at[pl.ds(s * RPT + k * DC, DC)], asem).wait()
    plsc.subcore_barrier()
    fill(1.0)

    def idx_copy(t, ib):
        return pltpu.make_async_copy(
            edges.at[1, pl.ds(w * DCH + 8 * t, 8)], dstb.at[ib], isem)

    def add_copy(ib, u):
        return pltpu.make_async_copy(
            ones, deg_tbl.at[dstb.at[ib].at[u]], asem)

    idx_copy(0, 0).start()
    idx_copy(0, 0).wait()

    def group(t, ib):
        @pl.when(t >= 2)
        def _():
            for u in range(8):
                add_copy((ib + 1) % 3, u).wait()

        @pl.when(t + 1 < DNG)
        def _():
            idx_copy(t + 1, (ib + 1) % 3).start()

        @pl.when(t >= 1)
        def _():
            idx_copy(t, ib).wait()
        for u in range(8):
            add_copy(ib, u).start(add=True)

    def triple(i, _):
        group(3 * i, 0)
        group(3 * i + 1, 1)
        group(3 * i + 2, 2)
        return 0
    lax.fori_loop(0, DNG // 3, triple, 0)
    group(DNG - 1, (DNG - 1) % 3)
    for u in range(8):
        add_copy((DNG - 2) % 3, u).wait()
        add_copy((DNG - 1) % 3, u).wait()

    plsc.subcore_barrier()
    pltpu.sync_copy(deg_tbl.at[pl.ds(s * RPT, RPT)],
                    out.at[c, pl.ds(s * RPT, RPT)])


@functools.partial(
    pl.kernel,
    out_type=jax.ShapeDtypeStruct((NC, NP, D), jnp.float32),
    mesh=_MESH,
    scratch_types=[
        pltpu.VMEM_SHARED((NP, D), jnp.float32),
        pltpu.VMEM((2, 8, SC_), jnp.int32),
        pltpu.VMEM((2, 8, SC_), jnp.int32),
        pltpu.VMEM((4, SC_, D), jnp.float32),
        [pltpu.SemaphoreType.DMA] * 4,
        [pltpu.SemaphoreType.DMA] * 4,
        pltpu.SemaphoreType.DMA,
    ],
)
def _sc_scatter(edges, y, out, acc, srcb, dstb, rows, gsems, ssems, isem):
    c = lax.axis_index("c")
    s = lax.axis_index("s")
    w = c * NS + s

    def fill_zeros(i, _):
        for k in range(D // 16):
            rows[0, i, pl.ds(k * 16, 16)] = jnp.zeros((16,), jnp.float32)
        return 0
    lax.fori_loop(0, SC_, fill_zeros, 0)
    for k in range(RPT // SC_):
        pltpu.make_async_copy(
            rows.at[0], acc.at[pl.ds(s * RPT + k * SC_, SC_)], isem).start()
    for k in range(RPT // SC_):
        pltpu.make_async_copy(
            rows.at[0], acc.at[pl.ds(s * RPT + k * SC_, SC_)], isem).wait()
    plsc.subcore_barrier()

    def idx_copies(t, ib):
        return (pltpu.make_async_copy(
                    edges.at[0, pl.ds(w * SCH + 8 * t, 8)], srcb.at[ib], isem),
                pltpu.make_async_copy(
                    edges.at[1, pl.ds(w * SCH + 8 * t, 8)], dstb.at[ib], isem))

    def g_copy(ib, u, b):
        return pltpu.make_async_copy(
            y.at[srcb.at[ib].at[u]], rows.at[b], gsems[b])

    def s_copy(ib, u, b):
        return pltpu.make_async_copy(
            rows.at[b], acc.at[dstb.at[ib].at[u]], ssems[b])

    for cp in idx_copies(0, 0):
        cp.start()
    for cp in idx_copies(0, 0):
        cp.wait()
    g_copy(0, 0, 0).start()
    g_copy(0, 1, 1).start()

    def group(t, ib):
        for u in range(8):
            j = 8 * t + u
            b = u % 4
            b2 = (u + 2) % 4
            pib = ib ^ 1 if u < 2 else ib

            @pl.when(j >= 2)
            def _():
                s_copy(pib, (u + 6) % 8, b2).wait()
            if u == 2:
                @pl.when(t + 1 < SNG)
                def _():
                    for cp in idx_copies(t + 1, ib ^ 1):
                        cp.start()
            nib = ib ^ 1 if u >= 6 else ib

            @pl.when(j + 2 < SCH)
            def _():
                if u == 6:
                    for cp in idx_copies(t + 1, ib ^ 1):
                        cp.wait()
                g_copy(nib, (u + 2) % 8, b2).start()
            g_copy(ib, u, b).wait()
            s_copy(ib, u, b).start(add=True)

    def pair(i, _):
        group(2 * i, 0)
        group(2 * i + 1, 1)
        return 0
    lax.fori_loop(0, SNG // 2, pair, 0)
    s_copy(1, 6, 2).wait()
    s_copy(1, 7, 3).wait()

    plsc.subcore_barrier()
    pltpu.sync_copy(acc.at[pl.ds(s * RPT, RPT)],
                    out.at[c, pl.ds(s * RPT, RPT)])


def _tc_matmul_body(x_ref, w_ref, xw_ref):
    xw_ref[...] = jnp.dot(x_ref[...], w_ref[...],
                          preferred_element_type=jnp.float32)


def _tc_matmul(x, conv_W):
    BN = 2000
    return pl.pallas_call(
        _tc_matmul_body,
        grid=(N // BN,),
        in_specs=[
            pl.BlockSpec((BN, D), lambda i: (i, 0)),
            pl.BlockSpec((D, D), lambda i: (0, 0)),
        ],
        out_specs=pl.BlockSpec((BN, D), lambda i: (i, 0)),
        out_shape=jax.ShapeDtypeStruct((N, D), jnp.float32),
    )(x, conv_W)


def _tc_scale_body(xw_ref, degp_ref, y_ref, dinv_ref):
    deg = degp_ref[0, :, 0:1] + degp_ref[1, :, 0:1] + 1.0
    dinv = lax.rsqrt(deg)
    y_ref[...] = dinv * xw_ref[...]
    dinv_ref[...] = dinv


def _tc_scale(xw, deg_parts):
    BN = 2000
    return pl.pallas_call(
        _tc_scale_body,
        grid=(N // BN,),
        in_specs=[
            pl.BlockSpec((BN, D), lambda i: (i, 0)),
            pl.BlockSpec((NC, BN, 16), lambda i: (0, i, 0)),
        ],
        out_specs=[
            pl.BlockSpec((BN, D), lambda i: (i, 0)),
            pl.BlockSpec((BN, 1), lambda i: (i, 0)),
        ],
        out_shape=[
            jax.ShapeDtypeStruct((N, D), jnp.float32),
            jax.ShapeDtypeStruct((N, 1), jnp.float32),
        ],
    )(xw, deg_parts)


def _tc_head_body(x_ref, y_ref, dinv_ref, acc_ref, cb_ref,
                  w1_ref, b1_ref, w2_ref, b2_ref, w3_ref, b3_ref, out_ref):
    g = dinv_ref[...] * (acc_ref[0] + acc_ref[1] + y_ref[...]) + cb_ref[...]
    h = jnp.maximum(g, 0.0) + x_ref[...]
    h1 = jnp.maximum(
        jnp.dot(h, w1_ref[...], preferred_element_type=jnp.float32)
        + b1_ref[...], 0.0)
    h2 = jnp.maximum(
        jnp.dot(h1, w2_ref[...], preferred_element_type=jnp.float32)
        + b2_ref[...], 0.0)
    out_ref[...] = (
        jnp.dot(h2, w3_ref[...], preferred_element_type=jnp.float32)
        + b3_ref[...])


def _tc_head(x, y, dinv, acc, conv_b, lin1_W, lin1_b, lin2_W, lin2_b,
             lin3_W, lin3_b):
    BN = 2000
    full = lambda shape: pl.BlockSpec(shape, lambda i: tuple(0 for _ in shape))
    return pl.pallas_call(
        _tc_head_body,
        grid=(N // BN,),
        in_specs=[
            pl.BlockSpec((BN, D), lambda i: (i, 0)),
            pl.BlockSpec((BN, D), lambda i: (i, 0)),
            pl.BlockSpec((BN, 1), lambda i: (i, 0)),
            pl.BlockSpec((NC, BN, D), lambda i: (0, i, 0)),
            full((1, D)),
            full((D, M)),
            full((1, M)),
            full((M, M)),
            full((1, M)),
            full((M, 1)),
            full((1, 1)),
        ],
        out_specs=pl.BlockSpec((BN, 1), lambda i: (i, 0)),
        out_shape=jax.ShapeDtypeStruct((N, 1), jnp.float32),
    )(x, y, dinv, acc, conv_b.reshape(1, D), lin1_W, lin1_b.reshape(1, M),
      lin2_W, lin2_b.reshape(1, M), lin3_W, lin3_b.reshape(1, 1))


def kernel(x, edge_index, conv_W, conv_b, lin1_W, lin1_b, lin2_W, lin2_b,
           lin3_W, lin3_b):
    ar = jnp.arange(EPAD - E, dtype=jnp.int32)
    pad_src = (ar * 997) % N
    pad_dst = N + (ar % (NP - N))
    epad = jnp.concatenate([edge_index, jnp.stack([pad_src, pad_dst])], axis=1)
    xw = _tc_matmul(x, conv_W)
    deg_parts = _sc_deg(epad.reshape(2, NW * DCH, DC))
    y, dinv = _tc_scale(xw, deg_parts)
    acc = _sc_scatter(epad.reshape(2, NW * SCH, SC_), y)
    return _tc_head(x, y, dinv, acc, conv_b, lin1_W, lin1_b,
                    lin2_W, lin2_b, lin3_W, lin3_b)
